# baseline (speedup 1.0000x reference)
import functools

import jax
import jax.numpy as jnp
from jax import lax
from jax.experimental import pallas as pl
from jax.experimental.pallas import tpu as pltpu
from jax.experimental.pallas import tpu_sc as plsc

N = 10000
T = 24
SD = 12
DDTD = 16
H = 128
HD2 = 64
NH = 4
HDIM = 32
L = 3
P = 24

NC = 2
NS = 16
NW = NC * NS
CH = 64
GRP = 16
NPAD = 10240
RPT = NPAD // NS


def _ln(x, g, b):
    m = jnp.mean(x, axis=-1, keepdims=True)
    v = jnp.mean((x - m) ** 2, axis=-1, keepdims=True)
    return (x - m) * lax.rsqrt(v + 1e-5) * g + b


def _encoder_body(dyn_ref, static_ref, dpWT_ref, dpb_ref, WihT_ref, WhhT_ref,
                  lstmb_ref, fusWT_ref, fusb_ref, femb_ref, WqT_ref, bq_ref,
                  WkT_ref, bk_ref, WvT_ref, bv_ref, WoT_ref, bo_ref,
                  l1qg_ref, l1qb_ref, l1kg_ref, l1kb_ref, l2g_ref, l2b_ref,
                  W1T_ref, b1_ref, W2T_ref, b2_ref, G_ref, GT_ref, out_ref):
    dyn = dyn_ref[...]
    Bn = dyn.shape[0]
    WihT = WihT_ref[...]
    WhhT = WhhT_ref[...]
    lstmb = lstmb_ref[...]
    dpWT = dpWT_ref[...]
    dpb = dpb_ref[...]
    h = jnp.zeros((Bn, HD2), jnp.float32)
    c = jnp.zeros((Bn, HD2), jnp.float32)
    for t in range(T):
        dt = dyn[:, t * DDTD:(t + 1) * DDTD] @ dpWT + dpb
        g = dt @ WihT + h @ WhhT + lstmb
        i = jax.nn.sigmoid(g[:, :HD2])
        f = jax.nn.sigmoid(g[:, HD2:2 * HD2])
        gg = jnp.tanh(g[:, 2 * HD2:3 * HD2])
        o = jax.nn.sigmoid(g[:, 3 * HD2:])
        c = f * c + i * gg
        h = o * jnp.tanh(c)
    dyn_out = h @ fusWT_ref[...] + fusb_ref[...]

    qn = _ln(dyn_out, l1qg_ref[...], l1qb_ref[...])
    qp = qn @ WqT_ref[...] + bq_ref[...]
    G = G_ref[...]
    GT = GT_ref[...]
    scale = 1.0 / (HDIM ** 0.5)
    scs = []
    vps = []
    for s in range(SD):
        tok = static_ref[:, s:s + 1] * femb_ref[s][None, :]
        kn = _ln(tok, l1kg_ref[...], l1kb_ref[...])
        kp = kn @ WkT_ref[...] + bk_ref[...]
        vp = kn @ WvT_ref[...] + bv_ref[...]
        scs.append(((qp * kp) @ G) * scale)
        vps.append(vp)
    m = scs[0]
    for s in range(1, SD):
        m = jnp.maximum(m, scs[s])
    es = [jnp.exp(sc - m) for sc in scs]
    den = es[0]
    for s in range(1, SD):
        den = den + es[s]
    inv = 1.0 / den
    ao = jnp.zeros((Bn, H), jnp.float32)
    for s in range(SD):
        ao = ao + vps[s] * ((es[s] * inv) @ GT)
    ao = ao @ WoT_ref[...] + bo_ref[...]
    xf = dyn_out + ao
    hln = _ln(xf, l2g_ref[...], l2b_ref[...])
    pre = hln @ W1T_ref[...] + b1_ref[...]
    f1 = pre * 0.5 * (1.0 + lax.erf(pre * (2.0 ** -0.5)))
    out_ref[...] = xf + f1 @ W2T_ref[...] + b2_ref[...]


def _full_spec(shape):
    nd = len(shape)
    return pl.BlockSpec(shape, lambda i: (0,) * nd)


def _encoder_call(dyn2, static, weights, block):
    grid = (N // block,)
    in_specs = [
        pl.BlockSpec((block, T * DDTD), lambda i: (i, 0)),
        pl.BlockSpec((block, SD), lambda i: (i, 0)),
    ] + [_full_spec(w.shape) for w in weights]
    return pl.pallas_call(
        _encoder_body,
        grid=grid,
        in_specs=in_specs,
        out_specs=pl.BlockSpec((block, H), lambda i: (i, 0)),
        out_shape=jax.ShapeDtypeStruct((N, H), jnp.float32),
    )(dyn2, static, *weights)


def _sc_mesh():
    return plsc.VectorSubcoreMesh(core_axis_name="c", subcore_axis_name="s",
                                  num_cores=NC, num_subcores=NS)


def _make_agg(ngrp):
    out_type = [jax.ShapeDtypeStruct((NC, NPAD, H), jnp.float32)]
    scratch = [
        pltpu.VMEM((GRP, CH), jnp.int32),
        pltpu.VMEM((GRP, CH), jnp.int32),
        pltpu.VMEM((CH, H), jnp.float32),
        pltpu.VMEM((CH, H), jnp.float32),
        pltpu.VMEM_SHARED((NPAD, H), jnp.float32),
        pltpu.SemaphoreType.DMA,
        pltpu.SemaphoreType.DMA,
        pltpu.SemaphoreType.DMA,
    ]

    def body(h, src4, dst4, z128, agg_out,
             src_v, dst_v, rows_a, rows_b, agg_sh, sem_g, sem_s0, sem_s1):
        c = lax.axis_index("c")
        s = lax.axis_index("s")
        wid = s * NC + c
        pltpu.sync_copy(z128, agg_sh.at[pl.ds(s * RPT, RPT)])
        plsc.subcore_barrier()
        bufs = (rows_a, rows_b)
        ssems = (sem_s0, sem_s1)

        def group(g, carry):
            pltpu.sync_copy(src4.at[wid, g], src_v)
            pltpu.sync_copy(dst4.at[wid, g], dst_v)
            gat = pltpu.async_copy(h.at[src_v.at[0]], bufs[0], sem_g)
            scat = [None, None]
            for b in range(GRP):
                cur = b % 2
                gat.wait()
                scat[cur] = pltpu.async_copy(
                    bufs[cur], agg_sh.at[dst_v.at[b]], ssems[cur], add=True)
                if b + 1 < GRP:
                    nxt = (b + 1) % 2
                    if scat[nxt] is not None:
                        scat[nxt].wait()
                    gat = pltpu.async_copy(
                        h.at[src_v.at[b + 1]], bufs[nxt], sem_g)
            scat[0].wait()
            scat[1].wait()
            return carry

        lax.fori_loop(0, ngrp, group, 0)
        plsc.subcore_barrier()
        pltpu.sync_copy(agg_sh.at[pl.ds(s * RPT, RPT)],
                        agg_out.at[c, pl.ds(s * RPT, RPT)])

    return pl.kernel(body, out_type=out_type, mesh=_sc_mesh(),
                     scratch_types=scratch)


def _make_deg(ngrp):
    out_type = [jax.ShapeDtypeStruct((NC, NPAD, H), jnp.float32)]
    scratch = [
        pltpu.VMEM((GRP, CH), jnp.int32),
        pltpu.VMEM((CH, H), jnp.float32),
        pltpu.VMEM_SHARED((NPAD, H), jnp.float32),
        pltpu.SemaphoreType.DMA,
    ]

    def body(dst4, z128, ones_h, deg_out, dst_v, ones_v, deg_sh, sem):
        c = lax.axis_index("c")
        s = lax.axis_index("s")
        wid = s * NC + c
        pltpu.sync_copy(z128, deg_sh.at[pl.ds(s * RPT, RPT)])
        pltpu.sync_copy(ones_h, ones_v)
        plsc.subcore_barrier()

        def group(g, carry):
            pltpu.sync_copy(dst4.at[wid, g], dst_v)
            for b in range(GRP):
                pltpu.sync_copy(ones_v, deg_sh.at[dst_v.at[b]], add=True)
            return carry

        lax.fori_loop(0, ngrp, group, 0)
        plsc.subcore_barrier()
        pltpu.sync_copy(deg_sh.at[pl.ds(s * RPT, RPT)],
                        deg_out.at[c, pl.ds(s * RPT, RPT)])

    return pl.kernel(body, out_type=out_type, mesh=_sc_mesh(),
                     scratch_types=scratch)


def _sage_body_mid(h_ref, p0_ref, p1_ref, d0_ref, d1_ref, WlT_ref, bl_ref,
                   WrT_ref, g_ref, b_ref, out_ref):
    deg = jnp.maximum(d0_ref[:, 0:1] + d1_ref[:, 0:1], 1.0)
    agg = (p0_ref[...] + p1_ref[...]) / deg
    h = h_ref[...]
    o = agg @ WlT_ref[...] + bl_ref[...] + h @ WrT_ref[...]
    o = _ln(o, g_ref[...], b_ref[...])
    out_ref[...] = jax.nn.relu(o) + h


def _sage_body_fin(h_ref, p0_ref, p1_ref, d0_ref, d1_ref, WlT_ref, bl_ref,
                   WrT_ref, g_ref, b_ref, outWT_ref, outb_ref, out_ref):
    deg = jnp.maximum(d0_ref[:, 0:1] + d1_ref[:, 0:1], 1.0)
    agg = (p0_ref[...] + p1_ref[...]) / deg
    h = h_ref[...]
    o = agg @ WlT_ref[...] + bl_ref[...] + h @ WrT_ref[...]
    o = _ln(o, g_ref[...], b_ref[...])
    hn = jax.nn.relu(o) + h
    out_ref[...] = hn @ outWT_ref[...] + outb_ref[...]


def _sage_call(final, h, p0, p1, d0, d1, weights, block):
    grid = (N // block,)
    in_specs = [
        pl.BlockSpec((block, H), lambda i: (i, 0)),
        pl.BlockSpec((block, H), lambda i: (i, 0)),
        pl.BlockSpec((block, H), lambda i: (i, 0)),
        pl.BlockSpec((block, H), lambda i: (i, 0)),
        pl.BlockSpec((block, H), lambda i: (i, 0)),
    ] + [_full_spec(w.shape) for w in weights]
    if final:
        body = _sage_body_fin
        out_spec = pl.BlockSpec((block, P), lambda i: (i, 0))
        out_shape = jax.ShapeDtypeStruct((N, P), jnp.float32)
    else:
        body = _sage_body_mid
        out_spec = pl.BlockSpec((block, H), lambda i: (i, 0))
        out_shape = jax.ShapeDtypeStruct((N, H), jnp.float32)
    return pl.pallas_call(
        body,
        grid=grid,
        in_specs=in_specs,
        out_specs=out_spec,
        out_shape=out_shape,
    )(h, p0, p1, d0, d1, *weights)


def kernel(x, params, edge_index):
    p = params
    static = x[:, -1, :SD]
    dyn2 = x[:, :, SD:].reshape(N, T * DDTD)

    E = edge_index.shape[1]
    slab = CH * GRP
    ept = ((E + NW * slab - 1) // (NW * slab)) * slab
    epad = ept * NW
    ngrp = ept // slab
    src = jnp.concatenate(
        [edge_index[0], jnp.zeros((epad - E,), jnp.int32)]).reshape(
            NW, ngrp, GRP, CH)
    dst = jnp.concatenate(
        [edge_index[1], jnp.full((epad - E,), N, jnp.int32)]).reshape(
            NW, ngrp, GRP, CH)
    z128 = jnp.zeros((RPT, H), jnp.float32)
    ones_h = jnp.ones((CH, H), jnp.float32)

    G = jnp.repeat(jnp.eye(NH, dtype=jnp.float32), HDIM, axis=0)

    enc_weights = [
        p['dp_W'].T, p['dp_b'],
        p['lstm_Wih'].T, p['lstm_Whh'].T, p['lstm_bih'] + p['lstm_bhh'],
        p['fus_dW'].T, p['fus_db'],
        p['feat_emb'][0],
        p['attn_inW'][:H].T, p['attn_inb'][:H],
        p['attn_inW'][H:2 * H].T, p['attn_inb'][H:2 * H],
        p['attn_inW'][2 * H:].T, p['attn_inb'][2 * H:],
        p['attn_outW'].T, p['attn_outb'],
        p['ln1q_g'], p['ln1q_b'], p['ln1kv_g'], p['ln1kv_b'],
        p['ln2_g'], p['ln2_b'],
        p['ffn_W1'].T, p['ffn_b1'], p['ffn_W2'].T, p['ffn_b2'],
        G, G.T,
    ]
    h = _encoder_call(dyn2, static, enc_weights, block=1000)

    agg_call = _make_agg(ngrp)
    (deg2,) = _make_deg(ngrp)(dst, z128, ones_h)
    deg0, deg1 = deg2[0], deg2[1]
    for l in range(L):
        (agg2,) = agg_call(h, src, dst, z128)
        sage_weights = [
            p['sage_Wl'][l].T, p['sage_bl'][l], p['sage_Wr'][l].T,
            p['nrm_g'][l], p['nrm_b'][l],
        ]
        if l == L - 1:
            sage_weights += [p['out_W'].T, p['out_b']]
        h = _sage_call(l == L - 1, h, agg2[0], agg2[1], deg0, deg1,
                       sage_weights, block=1000)
    return h


# speedup vs baseline: 1.0634x; 1.0634x over previous
import functools

import jax
import jax.numpy as jnp
from jax import lax
from jax.experimental import pallas as pl
from jax.experimental.pallas import tpu as pltpu
from jax.experimental.pallas import tpu_sc as plsc

N = 10000
T = 24
SD = 12
DDTD = 16
H = 128
HD2 = 64
NH = 4
HDIM = 32
L = 3
P = 24

NC = 2
NS = 16
NW = NC * NS
CH = 64
GRP = 16
NPAD = 10240
RPT = NPAD // NS


def _ln(x, g, b):
    m = jnp.mean(x, axis=-1, keepdims=True)
    v = jnp.mean((x - m) ** 2, axis=-1, keepdims=True)
    return (x - m) * lax.rsqrt(v + 1e-5) * g + b


def _encoder_body(dyn_ref, static_ref, dpWT_ref, dpb_ref, WihT_ref, WhhT_ref,
                  lstmb_ref, fusWT_ref, fusb_ref, femb_ref, WqT_ref, bq_ref,
                  WkT_ref, bk_ref, WvT_ref, bv_ref, WoT_ref, bo_ref,
                  l1qg_ref, l1qb_ref, l1kg_ref, l1kb_ref, l2g_ref, l2b_ref,
                  W1T_ref, b1_ref, W2T_ref, b2_ref, G_ref, GT_ref, out_ref):
    dyn = dyn_ref[...]
    Bn = dyn.shape[0]
    WihT = WihT_ref[...]
    WhhT = WhhT_ref[...]
    lstmb = lstmb_ref[...]
    dpWT = dpWT_ref[...]
    dpb = dpb_ref[...]
    h = jnp.zeros((Bn, HD2), jnp.float32)
    c = jnp.zeros((Bn, HD2), jnp.float32)
    for t in range(T):
        dt = dyn[:, t * DDTD:(t + 1) * DDTD] @ dpWT + dpb
        g = dt @ WihT + h @ WhhT + lstmb
        i = jax.nn.sigmoid(g[:, :HD2])
        f = jax.nn.sigmoid(g[:, HD2:2 * HD2])
        gg = jnp.tanh(g[:, 2 * HD2:3 * HD2])
        o = jax.nn.sigmoid(g[:, 3 * HD2:])
        c = f * c + i * gg
        h = o * jnp.tanh(c)
    dyn_out = h @ fusWT_ref[...] + fusb_ref[...]

    qn = _ln(dyn_out, l1qg_ref[...], l1qb_ref[...])
    qp = qn @ WqT_ref[...] + bq_ref[...]
    G = G_ref[...]
    GT = GT_ref[...]
    scale = 1.0 / (HDIM ** 0.5)
    scs = []
    vps = []
    for s in range(SD):
        tok = static_ref[:, s:s + 1] * femb_ref[s][None, :]
        kn = _ln(tok, l1kg_ref[...], l1kb_ref[...])
        kp = kn @ WkT_ref[...] + bk_ref[...]
        vp = kn @ WvT_ref[...] + bv_ref[...]
        scs.append(((qp * kp) @ G) * scale)
        vps.append(vp)
    m = scs[0]
    for s in range(1, SD):
        m = jnp.maximum(m, scs[s])
    es = [jnp.exp(sc - m) for sc in scs]
    den = es[0]
    for s in range(1, SD):
        den = den + es[s]
    inv = 1.0 / den
    ao = jnp.zeros((Bn, H), jnp.float32)
    for s in range(SD):
        ao = ao + vps[s] * ((es[s] * inv) @ GT)
    ao = ao @ WoT_ref[...] + bo_ref[...]
    xf = dyn_out + ao
    hln = _ln(xf, l2g_ref[...], l2b_ref[...])
    pre = hln @ W1T_ref[...] + b1_ref[...]
    f1 = pre * 0.5 * (1.0 + lax.erf(pre * (2.0 ** -0.5)))
    out_ref[...] = xf + f1 @ W2T_ref[...] + b2_ref[...]


def _full_spec(shape):
    nd = len(shape)
    return pl.BlockSpec(shape, lambda i: (0,) * nd)


def _encoder_call(dyn2, static, weights, block):
    grid = (N // block,)
    in_specs = [
        pl.BlockSpec((block, T * DDTD), lambda i: (i, 0)),
        pl.BlockSpec((block, SD), lambda i: (i, 0)),
    ] + [_full_spec(w.shape) for w in weights]
    return pl.pallas_call(
        _encoder_body,
        grid=grid,
        in_specs=in_specs,
        out_specs=pl.BlockSpec((block, H), lambda i: (i, 0)),
        out_shape=jax.ShapeDtypeStruct((N, H), jnp.float32),
    )(dyn2, static, *weights)


def _sc_mesh():
    return plsc.VectorSubcoreMesh(core_axis_name="c", subcore_axis_name="s",
                                  num_cores=NC, num_subcores=NS)


def _make_agg(ngrp):
    out_type = [jax.ShapeDtypeStruct((NC, NPAD, H), jnp.float32)]
    scratch = [
        pltpu.VMEM((GRP, CH), jnp.int32),
        pltpu.VMEM((GRP, CH), jnp.int32),
        pltpu.VMEM((CH, H), jnp.float32),
        pltpu.VMEM((CH, H), jnp.float32),
        pltpu.VMEM_SHARED((NPAD, H), jnp.float32),
        pltpu.SemaphoreType.DMA,
        pltpu.SemaphoreType.DMA,
        pltpu.SemaphoreType.DMA,
    ]

    def body(h, src4, dst4, z128, agg_out,
             src_v, dst_v, rows_a, rows_b, agg_sh, sem_g, sem_s0, sem_s1):
        c = lax.axis_index("c")
        s = lax.axis_index("s")
        wid = s * NC + c
        pltpu.sync_copy(z128, agg_sh.at[pl.ds(s * RPT, RPT)])
        plsc.subcore_barrier()
        bufs = (rows_a, rows_b)
        ssems = (sem_s0, sem_s1)

        def group(g, carry):
            pltpu.sync_copy(src4.at[wid, g], src_v)
            pltpu.sync_copy(dst4.at[wid, g], dst_v)
            gat = [
                pltpu.async_copy(h.at[src_v.at[0]], bufs[0], sem_s0),
                pltpu.async_copy(h.at[src_v.at[1]], bufs[1], sem_s1),
            ]
            for b in range(GRP):
                cur = b % 2
                gat[cur].wait()
                pltpu.sync_copy(bufs[cur], agg_sh.at[dst_v.at[b]], add=True)
                if b + 2 < GRP:
                    gat[cur] = pltpu.async_copy(
                        h.at[src_v.at[b + 2]], bufs[cur],
                        (sem_s0, sem_s1)[cur])
            return carry

        lax.fori_loop(0, ngrp, group, 0)
        plsc.subcore_barrier()
        pltpu.sync_copy(agg_sh.at[pl.ds(s * RPT, RPT)],
                        agg_out.at[c, pl.ds(s * RPT, RPT)])

    return pl.kernel(body, out_type=out_type, mesh=_sc_mesh(),
                     scratch_types=scratch)


def _make_deg(ngrp):
    out_type = [jax.ShapeDtypeStruct((NC, NPAD, H), jnp.float32)]
    scratch = [
        pltpu.VMEM((GRP, CH), jnp.int32),
        pltpu.VMEM((CH, H), jnp.float32),
        pltpu.VMEM_SHARED((NPAD, H), jnp.float32),
        pltpu.SemaphoreType.DMA,
    ]

    def body(dst4, z128, ones_h, deg_out, dst_v, ones_v, deg_sh, sem):
        c = lax.axis_index("c")
        s = lax.axis_index("s")
        wid = s * NC + c
        pltpu.sync_copy(z128, deg_sh.at[pl.ds(s * RPT, RPT)])
        pltpu.sync_copy(ones_h, ones_v)
        plsc.subcore_barrier()

        def group(g, carry):
            pltpu.sync_copy(dst4.at[wid, g], dst_v)
            for b in range(GRP):
                pltpu.sync_copy(ones_v, deg_sh.at[dst_v.at[b]], add=True)
            return carry

        lax.fori_loop(0, ngrp, group, 0)
        plsc.subcore_barrier()
        pltpu.sync_copy(deg_sh.at[pl.ds(s * RPT, RPT)],
                        deg_out.at[c, pl.ds(s * RPT, RPT)])

    return pl.kernel(body, out_type=out_type, mesh=_sc_mesh(),
                     scratch_types=scratch)


def _sage_body_mid(h_ref, p0_ref, p1_ref, d0_ref, d1_ref, WlT_ref, bl_ref,
                   WrT_ref, g_ref, b_ref, out_ref):
    deg = jnp.maximum(d0_ref[:, 0:1] + d1_ref[:, 0:1], 1.0)
    agg = (p0_ref[...] + p1_ref[...]) / deg
    h = h_ref[...]
    o = agg @ WlT_ref[...] + bl_ref[...] + h @ WrT_ref[...]
    o = _ln(o, g_ref[...], b_ref[...])
    out_ref[...] = jax.nn.relu(o) + h


def _sage_body_fin(h_ref, p0_ref, p1_ref, d0_ref, d1_ref, WlT_ref, bl_ref,
                   WrT_ref, g_ref, b_ref, outWT_ref, outb_ref, out_ref):
    deg = jnp.maximum(d0_ref[:, 0:1] + d1_ref[:, 0:1], 1.0)
    agg = (p0_ref[...] + p1_ref[...]) / deg
    h = h_ref[...]
    o = agg @ WlT_ref[...] + bl_ref[...] + h @ WrT_ref[...]
    o = _ln(o, g_ref[...], b_ref[...])
    hn = jax.nn.relu(o) + h
    out_ref[...] = hn @ outWT_ref[...] + outb_ref[...]


def _sage_call(final, h, p0, p1, d0, d1, weights, block):
    grid = (N // block,)
    in_specs = [
        pl.BlockSpec((block, H), lambda i: (i, 0)),
        pl.BlockSpec((block, H), lambda i: (i, 0)),
        pl.BlockSpec((block, H), lambda i: (i, 0)),
        pl.BlockSpec((block, H), lambda i: (i, 0)),
        pl.BlockSpec((block, H), lambda i: (i, 0)),
    ] + [_full_spec(w.shape) for w in weights]
    if final:
        body = _sage_body_fin
        out_spec = pl.BlockSpec((block, P), lambda i: (i, 0))
        out_shape = jax.ShapeDtypeStruct((N, P), jnp.float32)
    else:
        body = _sage_body_mid
        out_spec = pl.BlockSpec((block, H), lambda i: (i, 0))
        out_shape = jax.ShapeDtypeStruct((N, H), jnp.float32)
    return pl.pallas_call(
        body,
        grid=grid,
        in_specs=in_specs,
        out_specs=out_spec,
        out_shape=out_shape,
    )(h, p0, p1, d0, d1, *weights)


def kernel(x, params, edge_index):
    p = params
    static = x[:, -1, :SD]
    dyn2 = x[:, :, SD:].reshape(N, T * DDTD)

    E = edge_index.shape[1]
    slab = CH * GRP
    ept = ((E + NW * slab - 1) // (NW * slab)) * slab
    epad = ept * NW
    ngrp = ept // slab
    src = jnp.concatenate(
        [edge_index[0], jnp.zeros((epad - E,), jnp.int32)]).reshape(
            NW, ngrp, GRP, CH)
    dst = jnp.concatenate(
        [edge_index[1], jnp.full((epad - E,), N, jnp.int32)]).reshape(
            NW, ngrp, GRP, CH)
    z128 = jnp.zeros((RPT, H), jnp.float32)
    ones_h = jnp.ones((CH, H), jnp.float32)

    G = jnp.repeat(jnp.eye(NH, dtype=jnp.float32), HDIM, axis=0)

    enc_weights = [
        p['dp_W'].T, p['dp_b'],
        p['lstm_Wih'].T, p['lstm_Whh'].T, p['lstm_bih'] + p['lstm_bhh'],
        p['fus_dW'].T, p['fus_db'],
        p['feat_emb'][0],
        p['attn_inW'][:H].T, p['attn_inb'][:H],
        p['attn_inW'][H:2 * H].T, p['attn_inb'][H:2 * H],
        p['attn_inW'][2 * H:].T, p['attn_inb'][2 * H:],
        p['attn_outW'].T, p['attn_outb'],
        p['ln1q_g'], p['ln1q_b'], p['ln1kv_g'], p['ln1kv_b'],
        p['ln2_g'], p['ln2_b'],
        p['ffn_W1'].T, p['ffn_b1'], p['ffn_W2'].T, p['ffn_b2'],
        G, G.T,
    ]
    h = _encoder_call(dyn2, static, enc_weights, block=1000)

    agg_call = _make_agg(ngrp)
    (deg2,) = _make_deg(ngrp)(dst, z128, ones_h)
    deg0, deg1 = deg2[0], deg2[1]
    for l in range(L):
        (agg2,) = agg_call(h, src, dst, z128)
        sage_weights = [
            p['sage_Wl'][l].T, p['sage_bl'][l], p['sage_Wr'][l].T,
            p['nrm_g'][l], p['nrm_b'][l],
        ]
        if l == L - 1:
            sage_weights += [p['out_W'].T, p['out_b']]
        h = _sage_call(l == L - 1, h, agg2[0], agg2[1], deg0, deg1,
                       sage_weights, block=1000)
    return h
